# baseline (device time: 47145 ns/iter reference)
import jax
import jax.numpy as jnp
from jax import lax
from jax.experimental import pallas as pl
from jax.experimental.pallas import tpu as pltpu

N_DEV = 4
M_PER = 2048
CH = M_PER // N_DEV
HH = CH // 2
S = 4
SH = HH // S
K = 1024
N = 1024
FWD, BWD = 0, 1


def kernel(t, W):
    def body(t_ref, w_ref, out_ref, top_ref, bot_ref, w_bf_ref, rs_recv_ref,
             rs_send_sems, rs_recv_sems, ag_send_sems, ag_recv_sems):
        p = lax.axis_index("i")
        left = (p + N_DEV - 1) % N_DEV
        right = (p + 1) % N_DEV

        barrier_sem = pltpu.get_barrier_semaphore()
        for nbr in (left, right):
            pl.semaphore_signal(
                barrier_sem, inc=1,
                device_id=(nbr,), device_id_type=pl.DeviceIdType.MESH,
            )
        pl.semaphore_wait(barrier_sem, 2)

        def stage(c):
            for j in range(S):
                top_ref[c, j] = t_ref[
                    pl.ds(c * CH + j * SH, SH), :].astype(jnp.bfloat16)
                bot_ref[c, j] = t_ref[
                    pl.ds(c * CH + HH + j * SH, SH), :].astype(jnp.bfloat16)

        def rs_send_chunk(d, s):
            return (p - s + N_DEV) % N_DEV if d == FWD else (p + s) % N_DEV

        def rs_recv_chunk(d, s):
            return (p - s - 1 + N_DEV) % N_DEV if d == FWD \
                else (p + s + 1) % N_DEV

        def rs_rdma(d, s, j):
            buf = top_ref if d == FWD else bot_ref
            return pltpu.make_async_remote_copy(
                src_ref=buf.at[rs_send_chunk(d, s), j],
                dst_ref=rs_recv_ref.at[d, s, j],
                send_sem=rs_send_sems.at[d, s, j],
                recv_sem=rs_recv_sems.at[d, s, j],
                device_id=(right if d == FWD else left,),
                device_id_type=pl.DeviceIdType.MESH,
            )

        def ag_rows(d, h):
            if d == FWD:
                o = (p + 1 - h + N_DEV) % N_DEV
                return o * CH
            o = (p - 1 + h + N_DEV) % N_DEV
            return o * CH + HH

        def ag_rdma(d, h, j):
            rows = ag_rows(d, h)
            sl = out_ref.at[pl.ds(rows + j * SH, SH), :]
            return pltpu.make_async_remote_copy(
                src_ref=sl, dst_ref=sl,
                send_sem=ag_send_sems.at[d, h, j],
                recv_sem=ag_recv_sems.at[d, h, j],
                device_id=(right if d == FWD else left,),
                device_id_type=pl.DeviceIdType.MESH,
            )

        stage(p)
        for d in (FWD, BWD):
            for j in range(S):
                rs_rdma(d, 0, j).start()
        for o in range(1, N_DEV):
            stage((p + o) % N_DEV)
        w_bf_ref[...] = w_ref[...].astype(jnp.bfloat16)

        own = {FWD: (p + 1) % N_DEV, BWD: (p - 1 + N_DEV) % N_DEV}

        for s in range(N_DEV - 1):
            for j in range(S):
                for d in (FWD, BWD):
                    buf = top_ref if d == FWD else bot_ref
                    rs_rdma(d, s, j).wait_recv()
                    rc = rs_recv_chunk(d, s)
                    buf[rc, j] = buf[rc, j] + rs_recv_ref[d, s, j]
                    if s < N_DEV - 2:
                        rs_rdma(d, s + 1, j).start()
                    else:
                        acc = lax.dot_general(
                            buf[own[d], j], w_bf_ref[...],
                            dimension_numbers=(((1,), (0,)), ((), ())),
                            preferred_element_type=jnp.float32,
                        )
                        out_ref[pl.ds(ag_rows(d, 0) + j * SH, SH), :] = (
                            acc.astype(jnp.bfloat16))
                        ag_rdma(d, 0, j).start()

        for h in range(N_DEV - 1):
            for j in range(S):
                for d in (FWD, BWD):
                    ag_rdma(d, h, j).wait_recv()
                    if h < N_DEV - 2:
                        ag_rdma(d, h + 1, j).start()

        for s in range(N_DEV - 1):
            for j in range(S):
                for d in (FWD, BWD):
                    rs_rdma(d, s, j).wait_send()
                    ag_rdma(d, s, j).wait_send()

    return pl.pallas_call(
        body,
        out_shape=jax.ShapeDtypeStruct((M_PER, N), jnp.bfloat16),
        in_specs=[
            pl.BlockSpec(memory_space=pltpu.VMEM),
            pl.BlockSpec(memory_space=pltpu.VMEM),
        ],
        out_specs=pl.BlockSpec(memory_space=pltpu.VMEM),
        scratch_shapes=[
            pltpu.VMEM((N_DEV, S, SH, K), jnp.bfloat16),
            pltpu.VMEM((N_DEV, S, SH, K), jnp.bfloat16),
            pltpu.VMEM((K, N), jnp.bfloat16),
            pltpu.VMEM((2, N_DEV - 1, S, SH, K), jnp.bfloat16),
            pltpu.SemaphoreType.DMA((2, N_DEV - 1, S)),
            pltpu.SemaphoreType.DMA((2, N_DEV - 1, S)),
            pltpu.SemaphoreType.DMA((2, N_DEV - 1, S)),
            pltpu.SemaphoreType.DMA((2, N_DEV - 1, S)),
        ],
        compiler_params=pltpu.CompilerParams(collective_id=0),
    )(t, W)


# device time: 46673 ns/iter; 1.0101x vs baseline; 1.0101x over previous
import jax
import jax.numpy as jnp
from jax import lax
from jax.experimental import pallas as pl
from jax.experimental.pallas import tpu as pltpu

N_DEV = 4
M_PER = 2048
CH = M_PER // N_DEV
HH = CH // 2
S = 2
SH = HH // S
K = 1024
N = 1024
FWD, BWD = 0, 1


def kernel(t, W):
    def body(t_ref, w_ref, out_ref, top_ref, bot_ref, w_bf_ref, rs_recv_ref,
             rs_send_sems, rs_recv_sems, ag_send_sems, ag_recv_sems):
        p = lax.axis_index("i")
        left = (p + N_DEV - 1) % N_DEV
        right = (p + 1) % N_DEV

        barrier_sem = pltpu.get_barrier_semaphore()
        for nbr in (left, right):
            pl.semaphore_signal(
                barrier_sem, inc=1,
                device_id=(nbr,), device_id_type=pl.DeviceIdType.MESH,
            )
        pl.semaphore_wait(barrier_sem, 2)

        def stage(c):
            for j in range(S):
                top_ref[c, j] = t_ref[
                    pl.ds(c * CH + j * SH, SH), :].astype(jnp.bfloat16)
                bot_ref[c, j] = t_ref[
                    pl.ds(c * CH + HH + j * SH, SH), :].astype(jnp.bfloat16)

        def rs_send_chunk(d, s):
            return (p - s + N_DEV) % N_DEV if d == FWD else (p + s) % N_DEV

        def rs_recv_chunk(d, s):
            return (p - s - 1 + N_DEV) % N_DEV if d == FWD \
                else (p + s + 1) % N_DEV

        def rs_rdma(d, s, j):
            buf = top_ref if d == FWD else bot_ref
            return pltpu.make_async_remote_copy(
                src_ref=buf.at[rs_send_chunk(d, s), j],
                dst_ref=rs_recv_ref.at[d, s, j],
                send_sem=rs_send_sems.at[d, s, j],
                recv_sem=rs_recv_sems.at[d, s, j],
                device_id=(right if d == FWD else left,),
                device_id_type=pl.DeviceIdType.MESH,
            )

        def ag_rows(d, h):
            if d == FWD:
                o = (p + 1 - h + N_DEV) % N_DEV
                return o * CH
            o = (p - 1 + h + N_DEV) % N_DEV
            return o * CH + HH

        def ag_rdma(d, h, j):
            rows = ag_rows(d, h)
            sl = out_ref.at[pl.ds(rows + j * SH, SH), :]
            return pltpu.make_async_remote_copy(
                src_ref=sl, dst_ref=sl,
                send_sem=ag_send_sems.at[d, h, j],
                recv_sem=ag_recv_sems.at[d, h, j],
                device_id=(right if d == FWD else left,),
                device_id_type=pl.DeviceIdType.MESH,
            )

        stage(p)
        for d in (FWD, BWD):
            for j in range(S):
                rs_rdma(d, 0, j).start()
        for o in range(1, N_DEV):
            stage((p + o) % N_DEV)
        w_bf_ref[...] = w_ref[...].astype(jnp.bfloat16)

        own = {FWD: (p + 1) % N_DEV, BWD: (p - 1 + N_DEV) % N_DEV}

        for s in range(N_DEV - 1):
            for j in range(S):
                for d in (FWD, BWD):
                    buf = top_ref if d == FWD else bot_ref
                    rs_rdma(d, s, j).wait_recv()
                    rc = rs_recv_chunk(d, s)
                    buf[rc, j] = buf[rc, j] + rs_recv_ref[d, s, j]
                    if s < N_DEV - 2:
                        rs_rdma(d, s + 1, j).start()
                    else:
                        acc = lax.dot_general(
                            buf[own[d], j], w_bf_ref[...],
                            dimension_numbers=(((1,), (0,)), ((), ())),
                            preferred_element_type=jnp.float32,
                        )
                        out_ref[pl.ds(ag_rows(d, 0) + j * SH, SH), :] = (
                            acc.astype(jnp.bfloat16))
                        ag_rdma(d, 0, j).start()

        for h in range(N_DEV - 1):
            for j in range(S):
                for d in (FWD, BWD):
                    ag_rdma(d, h, j).wait_recv()
                    if h < N_DEV - 2:
                        ag_rdma(d, h + 1, j).start()

        for s in range(N_DEV - 1):
            for j in range(S):
                for d in (FWD, BWD):
                    rs_rdma(d, s, j).wait_send()
                    ag_rdma(d, s, j).wait_send()

    return pl.pallas_call(
        body,
        out_shape=jax.ShapeDtypeStruct((M_PER, N), jnp.bfloat16),
        in_specs=[
            pl.BlockSpec(memory_space=pltpu.VMEM),
            pl.BlockSpec(memory_space=pltpu.VMEM),
        ],
        out_specs=pl.BlockSpec(memory_space=pltpu.VMEM),
        scratch_shapes=[
            pltpu.VMEM((N_DEV, S, SH, K), jnp.bfloat16),
            pltpu.VMEM((N_DEV, S, SH, K), jnp.bfloat16),
            pltpu.VMEM((K, N), jnp.bfloat16),
            pltpu.VMEM((2, N_DEV - 1, S, SH, K), jnp.bfloat16),
            pltpu.SemaphoreType.DMA((2, N_DEV - 1, S)),
            pltpu.SemaphoreType.DMA((2, N_DEV - 1, S)),
            pltpu.SemaphoreType.DMA((2, N_DEV - 1, S)),
            pltpu.SemaphoreType.DMA((2, N_DEV - 1, S)),
        ],
        compiler_params=pltpu.CompilerParams(collective_id=0),
    )(t, W)


# device time: 46612 ns/iter; 1.0114x vs baseline; 1.0013x over previous
import os

import jax
import jax.numpy as jnp
from jax import lax
from jax.experimental import pallas as pl
from jax.experimental.pallas import tpu as pltpu

SKIP_RS = os.environ.get("SKIP_RS") == "1"
SKIP_AG = os.environ.get("SKIP_AG") == "1"

N_DEV = 4
M_PER = 2048
CH = M_PER // N_DEV
HH = CH // 2
S = 2
SH = HH // S
K = 1024
N = 1024
FWD, BWD = 0, 1


def kernel(t, W):
    def body(t_ref, w_ref, out_ref, t_vmem_ref, top_ref, bot_ref, w_bf_ref,
             rs_recv_ref, t_copy_sems, rs_send_sems, rs_recv_sems,
             ag_send_sems, ag_recv_sems):
        p = lax.axis_index("i")
        left = (p + N_DEV - 1) % N_DEV
        right = (p + 1) % N_DEV

        use_rdma = not (SKIP_RS and SKIP_AG)
        if use_rdma:
            barrier_sem = pltpu.get_barrier_semaphore()
            for nbr in (left, right):
                pl.semaphore_signal(
                    barrier_sem, inc=1,
                    device_id=(nbr,), device_id_type=pl.DeviceIdType.MESH,
                )

        def t_copy(o):
            c = (p + o) % N_DEV
            return pltpu.make_async_copy(
                t_ref.at[pl.ds(c * CH, CH), :],
                t_vmem_ref.at[pl.ds(c * CH, CH), :],
                t_copy_sems.at[o],
            )

        for o in range(N_DEV):
            t_copy(o).start()

        def t_slab(d, c, j):
            base = c * CH + (0 if d == FWD else HH)
            return t_vmem_ref[pl.ds(base + j * SH, SH), :].astype(
                jnp.bfloat16)

        t_copy(0).wait()
        for j in range(S):
            top_ref[p, j] = t_slab(FWD, p, j)
            bot_ref[p, j] = t_slab(BWD, p, j)

        if use_rdma:
            pl.semaphore_wait(barrier_sem, 2)

        def rs_send_chunk(d, s):
            return (p - s + N_DEV) % N_DEV if d == FWD else (p + s) % N_DEV

        def rs_recv_chunk(d, s):
            return (p - s - 1 + N_DEV) % N_DEV if d == FWD \
                else (p + s + 1) % N_DEV

        def rs_rdma(d, s, j):
            buf = top_ref if d == FWD else bot_ref
            return pltpu.make_async_remote_copy(
                src_ref=buf.at[rs_send_chunk(d, s), j],
                dst_ref=rs_recv_ref.at[d, s, j],
                send_sem=rs_send_sems.at[d, s, j],
                recv_sem=rs_recv_sems.at[d, s, j],
                device_id=(right if d == FWD else left,),
                device_id_type=pl.DeviceIdType.MESH,
            )

        def ag_rows(d, h):
            if d == FWD:
                o = (p + 1 - h + N_DEV) % N_DEV
                return o * CH
            o = (p - 1 + h + N_DEV) % N_DEV
            return o * CH + HH

        def ag_rdma(d, h, j):
            rows = ag_rows(d, h)
            sl = out_ref.at[pl.ds(rows + j * SH, SH), :]
            return pltpu.make_async_remote_copy(
                src_ref=sl, dst_ref=sl,
                send_sem=ag_send_sems.at[d, h, j],
                recv_sem=ag_recv_sems.at[d, h, j],
                device_id=(right if d == FWD else left,),
                device_id_type=pl.DeviceIdType.MESH,
            )

        if not SKIP_RS:
            for d in (FWD, BWD):
                for j in range(S):
                    rs_rdma(d, 0, j).start()
        w_bf_ref[...] = w_ref[...].astype(jnp.bfloat16)
        for o in range(1, N_DEV):
            t_copy(o).wait()

        own = {FWD: (p + 1) % N_DEV, BWD: (p - 1 + N_DEV) % N_DEV}

        def mm_and_ag(d, j):
            buf = top_ref if d == FWD else bot_ref
            acc = lax.dot_general(
                buf[own[d], j], w_bf_ref[...],
                dimension_numbers=(((1,), (0,)), ((), ())),
                preferred_element_type=jnp.float32,
            )
            out_ref[pl.ds(ag_rows(d, 0) + j * SH, SH), :] = (
                acc.astype(jnp.bfloat16))
            if not SKIP_AG:
                ag_rdma(d, 0, j).start()

        if SKIP_RS:
            for j in range(S):
                for d in (FWD, BWD):
                    buf = top_ref if d == FWD else bot_ref
                    buf[own[d], j] = t_slab(d, own[d], j)
                    mm_and_ag(d, j)
        else:
            for s in range(N_DEV - 1):
                for j in range(S):
                    for d in (FWD, BWD):
                        buf = top_ref if d == FWD else bot_ref
                        rc = rs_recv_chunk(d, s)
                        local = t_slab(d, rc, j)
                        rs_rdma(d, s, j).wait_recv()
                        buf[rc, j] = local + rs_recv_ref[d, s, j]
                        if s < N_DEV - 2:
                            rs_rdma(d, s + 1, j).start()
                        else:
                            mm_and_ag(d, j)

        if not SKIP_AG:
            for h in range(N_DEV - 1):
                for j in range(S):
                    for d in (FWD, BWD):
                        ag_rdma(d, h, j).wait_recv()
                        if h < N_DEV - 2:
                            ag_rdma(d, h + 1, j).start()

        for s in range(N_DEV - 1):
            for j in range(S):
                for d in (FWD, BWD):
                    if not SKIP_RS:
                        rs_rdma(d, s, j).wait_send()
                    if not SKIP_AG:
                        ag_rdma(d, s, j).wait_send()

    return pl.pallas_call(
        body,
        out_shape=jax.ShapeDtypeStruct((M_PER, N), jnp.bfloat16),
        in_specs=[
            pl.BlockSpec(memory_space=pl.ANY),
            pl.BlockSpec(memory_space=pltpu.VMEM),
        ],
        out_specs=pl.BlockSpec(memory_space=pltpu.VMEM),
        scratch_shapes=[
            pltpu.VMEM((M_PER, K), jnp.float32),
            pltpu.VMEM((N_DEV, S, SH, K), jnp.bfloat16),
            pltpu.VMEM((N_DEV, S, SH, K), jnp.bfloat16),
            pltpu.VMEM((K, N), jnp.bfloat16),
            pltpu.VMEM((2, N_DEV - 1, S, SH, K), jnp.bfloat16),
            pltpu.SemaphoreType.DMA((N_DEV,)),
            pltpu.SemaphoreType.DMA((2, N_DEV - 1, S)),
            pltpu.SemaphoreType.DMA((2, N_DEV - 1, S)),
            pltpu.SemaphoreType.DMA((2, N_DEV - 1, S)),
            pltpu.SemaphoreType.DMA((2, N_DEV - 1, S)),
        ],
        compiler_params=(
            pltpu.CompilerParams(collective_id=0)
            if not (SKIP_RS and SKIP_AG) else pltpu.CompilerParams()
        ),
    )(t, W)
